# baseline (speedup 1.0000x reference)
import functools

import jax
import jax.numpy as jnp
from jax import lax
from jax.experimental import pallas as pl
from jax.experimental.pallas import tpu as pltpu
from jax.experimental.pallas import tpu_sc as plsc

E = 8
TOP_K = 2
D = 1024
FF = 3584
T = 2048
TK = T * TOP_K
B = 256
NB = (TK + E * B) // B
NPAD = NB * B

NW = 32
NC = 2


def _router_body(x_ref, wg_ref, i1_ref, i2_ref, wa_ref, wb_ref):
    x = x_ref[...]
    wg = wg_ref[...]
    logits = lax.dot_general(x, wg, (((1,), (1,)), ((), ())),
                             preferred_element_type=jnp.float32)
    ii = lax.broadcasted_iota(jnp.int32, (T, E), 1)
    m1 = jnp.max(logits, axis=1, keepdims=True)
    i1 = jnp.min(jnp.where(logits == m1, ii, E), axis=1, keepdims=True)
    masked = jnp.where(ii == i1, -jnp.inf, logits)
    m2 = jnp.max(masked, axis=1, keepdims=True)
    i2 = jnp.min(jnp.where(masked == m2, ii, E), axis=1, keepdims=True)
    wa = 1.0 / (1.0 + jnp.exp(m2 - m1))
    i1_ref[...] = i1
    i2_ref[...] = i2
    wa_ref[...] = wa
    wb_ref[...] = 1.0 - wa


def _router(x, w_gate):
    return pl.pallas_call(
        _router_body,
        out_shape=(
            jax.ShapeDtypeStruct((T, 1), jnp.int32),
            jax.ShapeDtypeStruct((T, 1), jnp.int32),
            jax.ShapeDtypeStruct((T, 1), jnp.float32),
            jax.ShapeDtypeStruct((T, 1), jnp.float32),
        ),
    )(x, w_gate)


def _dispatch_meta(i1, i2, wa, wb):
    e_f = jnp.stack([i1, i2], axis=1).reshape(TK)
    oh = (e_f[:, None] == jnp.arange(E, dtype=jnp.int32)[None, :]).astype(jnp.int32)
    csum = jnp.cumsum(oh, axis=0)
    counts = csum[-1]
    rank = jnp.take_along_axis(csum, e_f[:, None], axis=1)[:, 0] - 1
    padded = ((counts + B - 1) // B) * B
    gend = jnp.cumsum(padded)
    gstart = gend - padded
    dest = (gstart[e_f] + rank).astype(jnp.int32)
    tok = jnp.arange(TK, dtype=jnp.int32) // TOP_K
    src_tok = jnp.zeros((NPAD,), jnp.int32).at[dest].set(tok)
    w_f = jnp.stack([wa, wb], axis=1).reshape(TK)
    wsort = jnp.zeros((NPAD,), jnp.float32).at[dest].set(w_f)
    pos0 = dest[0::2]
    pos1 = dest[1::2]
    bidx = jnp.arange(NB, dtype=jnp.int32)
    ends_b = (gend // B).astype(jnp.int32)
    be_raw = jnp.sum((bidx[:, None] >= ends_b[None, :]).astype(jnp.int32),
                     axis=1)
    return src_tok, wsort, pos0, pos1, be_raw


GCH = 64


def _sc_gather(x, src_tok):
    mesh = plsc.VectorSubcoreMesh(core_axis_name="c", subcore_axis_name="s")
    rows_per_w = NPAD // NW

    @functools.partial(
        pl.kernel,
        mesh=mesh,
        out_type=jax.ShapeDtypeStruct((NPAD, D), jnp.float32),
        scratch_types=[
            pltpu.VMEM((GCH,), jnp.int32),
            pltpu.VMEM((GCH, D), jnp.float32),
            pltpu.SemaphoreType.DMA,
        ],
    )
    def k(x_hbm, idx_hbm, out_hbm, idx_v, rows_v, sem):
        wid = lax.axis_index("s") * NC + lax.axis_index("c")
        base = wid * rows_per_w
        for t in range(rows_per_w // GCH):
            b0 = base + t * GCH
            pltpu.sync_copy(idx_hbm.at[pl.ds(b0, GCH)], idx_v)
            pltpu.async_copy(x_hbm.at[idx_v], rows_v, sem).wait()
            pltpu.sync_copy(rows_v, out_hbm.at[pl.ds(b0, GCH)])

    return k(x, src_tok)


def _ffn_body(be_ref, x_ref, w1_ref, w3_ref, w2_ref, ws_ref, y_ref):
    b = pl.program_id(0)

    @pl.when(be_ref[b] < E)
    def _():
        xb = x_ref[...].astype(jnp.bfloat16)
        g = lax.dot_general(xb, w1_ref[0], (((1,), (1,)), ((), ())),
                            preferred_element_type=jnp.float32)
        u = lax.dot_general(xb, w3_ref[0], (((1,), (1,)), ((), ())),
                            preferred_element_type=jnp.float32)
        sg = 1.0 / (1.0 + jnp.exp(-g))
        h = (g * sg * u).astype(jnp.bfloat16)
        out = lax.dot_general(h, w2_ref[0], (((1,), (1,)), ((), ())),
                              preferred_element_type=jnp.float32)
        y_ref[...] = out * ws_ref[:, 0:1]


def _ffn(be_raw, x_sorted, wb1, wb3, wb2, w_bcast):
    def wsel(b, be):
        return jnp.minimum(be[b], E - 1)

    grid_spec = pltpu.PrefetchScalarGridSpec(
        num_scalar_prefetch=1,
        grid=(NB,),
        in_specs=[
            pl.BlockSpec((B, D), lambda b, be: (b, 0)),
            pl.BlockSpec((1, FF, D), lambda b, be: (wsel(b, be), 0, 0)),
            pl.BlockSpec((1, FF, D), lambda b, be: (wsel(b, be), 0, 0)),
            pl.BlockSpec((1, D, FF), lambda b, be: (wsel(b, be), 0, 0)),
            pl.BlockSpec((B, 128), lambda b, be: (b, 0)),
        ],
        out_specs=pl.BlockSpec((B, D), lambda b, be: (b, 0)),
    )
    return pl.pallas_call(
        _ffn_body,
        grid_spec=grid_spec,
        out_shape=jax.ShapeDtypeStruct((NPAD, D), jnp.float32),
        compiler_params=pltpu.CompilerParams(
            dimension_semantics=("arbitrary",)),
    )(be_raw, x_sorted, wb1, wb3, wb2, w_bcast)


CCH = 32


def _sc_combine(y, pos0, pos1):
    mesh = plsc.VectorSubcoreMesh(core_axis_name="c", subcore_axis_name="s")
    tok_per_w = T // NW

    @functools.partial(
        pl.kernel,
        mesh=mesh,
        out_type=jax.ShapeDtypeStruct((T, D), jnp.float32),
        scratch_types=[
            pltpu.VMEM((CCH,), jnp.int32),
            pltpu.VMEM((CCH,), jnp.int32),
            pltpu.VMEM((CCH, D), jnp.float32),
            pltpu.VMEM((CCH, D), jnp.float32),
            pltpu.SemaphoreType.DMA,
            pltpu.SemaphoreType.DMA,
        ],
    )
    def k(y_hbm, p0_hbm, p1_hbm, out_hbm, i0_v, i1_v, r0_v, r1_v, s0, s1):
        wid = lax.axis_index("s") * NC + lax.axis_index("c")
        base = wid * tok_per_w
        for t in range(tok_per_w // CCH):
            b0 = base + t * CCH
            pltpu.sync_copy(p0_hbm.at[pl.ds(b0, CCH)], i0_v)
            pltpu.sync_copy(p1_hbm.at[pl.ds(b0, CCH)], i1_v)
            cp0 = pltpu.async_copy(y_hbm.at[i0_v], r0_v, s0)
            cp1 = pltpu.async_copy(y_hbm.at[i1_v], r1_v, s1)
            cp0.wait()
            cp1.wait()

            def body(i, _):
                r = i // (D // 16)
                c = (i % (D // 16)) * 16
                r0_v[r, pl.ds(c, 16)] = (r0_v[r, pl.ds(c, 16)]
                                         + r1_v[r, pl.ds(c, 16)])
                return 0

            lax.fori_loop(0, CCH * (D // 16), body, 0)
            pltpu.sync_copy(r0_v, out_hbm.at[pl.ds(b0, CCH)])

    return k(y, pos0, pos1)


def kernel(hidden_states, w_gate, w1, w3, w2):
    orig_shape = hidden_states.shape
    x = hidden_states.reshape(T, D)
    i1, i2, wa, wb = _router(x, w_gate)
    src_tok, wsort, pos0, pos1, be_raw = _dispatch_meta(
        i1[:, 0], i2[:, 0], wa[:, 0], wb[:, 0])
    x_sorted = _sc_gather(x, src_tok)
    wb1 = w1.astype(jnp.bfloat16)
    wb3 = w3.astype(jnp.bfloat16)
    wb2 = w2.astype(jnp.bfloat16)
    w_bcast = jnp.broadcast_to(wsort[:, None], (NPAD, 128))
    y = _ffn(be_raw, x_sorted, wb1, wb3, wb2, w_bcast)
    final = _sc_combine(y, pos0, pos1)
    return final.reshape(orig_shape)
